# baseline (speedup 1.0000x reference)
import functools
import jax
import jax.numpy as jnp
import numpy as np
from jax import lax
from jax.experimental import pallas as pl
from jax.experimental.pallas import tpu as pltpu, tpu_sc as plsc

_BINS = 10
_EDGES32 = np.arange(_BINS + 1, dtype=np.float32) / np.float32(_BINS)
_LOGITS = [float(np.log(np.float64(e) / (1.0 - np.float64(e))))
           for e in _EDGES32[1:_BINS]]

_N = 4194304
_NW = 32
_SHARD = _N // _NW
_CHUNK = 8192
_NCHUNKS = _SHARD // _CHUNK
_L = 16
_ROWS = 24


def _sc_body(x_hbm, y_hbm, out_hbm, xa, ya, part, dsem):
    c = lax.axis_index("c")
    s = lax.axis_index("s")
    wid = s * 2 + c
    base = wid * _SHARD

    def chunk_step(k, accs):
        pltpu.sync_copy(x_hbm.at[pl.ds(base + k * _CHUNK, _CHUNK)], xa)
        pltpu.sync_copy(y_hbm.at[pl.ds(base + k * _CHUNK, _CHUNK)], ya)

        def vec_step(i, accs2):
            x_v = xa[pl.ds(i * _L, _L)]
            y_v = ya[pl.ds(i * _L, _L)]
            t = jnp.where(y_v == 0, x_v, -x_v)
            e = jnp.exp(-jnp.abs(t))
            z = e / (2.0 + e)
            z2 = z * z
            ln1p = 2.0 * z * (1.0 + z2 * (1.0 / 3.0 + z2 *
                              (0.2 + z2 * (1.0 / 7.0 + z2 * (1.0 / 9.0)))))
            pe = jnp.maximum(t, 0.0) + ln1p
            out = [accs2[0] + pe]
            for j in range(1, _BINS):
                mf = jnp.where(t >= _LOGITS[j - 1], 1.0, 0.0)
                out.append(accs2[j] + mf * pe)
            for j in range(1, _BINS):
                mf = jnp.where(t >= _LOGITS[j - 1], 1.0, 0.0)
                out.append(accs2[9 + j] + mf)
            return tuple(out)

        return lax.fori_loop(0, _CHUNK // _L, vec_step, accs)

    zero = jnp.zeros((_L,), jnp.float32)
    accs = lax.fori_loop(0, _NCHUNKS, chunk_step,
                         tuple(zero for _ in range(2 * _BINS - 1)))
    for j in range(2 * _BINS - 1):
        part[j, :] = accs[j]
    part[2 * _BINS - 1, :] = zero
    for j in range(2 * _BINS, _ROWS):
        part[j, :] = zero
    pltpu.sync_copy(part, out_hbm.at[wid])


def _tc_finish_body(p_ref, out_ref):
    p = jnp.sum(p_ref[...], axis=0)
    u = [jnp.sum(p[j * _L:(j + 1) * _L]) for j in range(_BINS)]
    tt = [jnp.float32(_N)] + [jnp.sum(p[(9 + j) * _L:(10 + j) * _L])
                              for j in range(1, _BINS)]
    num = jnp.float32(0.0)
    acc = jnp.float32(0.0)
    for b in range(_BINS):
        tb1 = jnp.float32(0.0) if b == _BINS - 1 else tt[b + 1]
        ub1 = jnp.float32(0.0) if b == _BINS - 1 else u[b + 1]
        cnt = tt[b] - tb1
        sv = u[b] - ub1
        pos = cnt > 0.0
        num = num + jnp.where(pos, 1.0, 0.0)
        acc = acc + jnp.where(pos, sv / jnp.maximum(cnt, 1.0), 0.0)
    out_ref[0, 0] = acc / jnp.maximum(num, 1.0)


def kernel(y_pred, y_true):
    n = y_pred.shape[0]
    x1 = y_pred.reshape(n)
    y1 = y_true.reshape(n).astype(jnp.int32)

    mesh = plsc.VectorSubcoreMesh(core_axis_name="c", subcore_axis_name="s")
    sc = functools.partial(
        pl.kernel,
        mesh=mesh,
        out_type=jax.ShapeDtypeStruct((_NW, _ROWS, _L), jnp.float32),
        scratch_types=[
            pltpu.VMEM((_CHUNK,), jnp.float32),
            pltpu.VMEM((_CHUNK,), jnp.int32),
            pltpu.VMEM((_ROWS, _L), jnp.float32),
            pltpu.SemaphoreType.DMA,
        ],
    )(_sc_body)
    partials = sc(x1, y1)

    out = pl.pallas_call(
        _tc_finish_body,
        in_specs=[pl.BlockSpec((_NW, _ROWS * _L), lambda: (0, 0))],
        out_specs=pl.BlockSpec(memory_space=pltpu.SMEM),
        out_shape=jax.ShapeDtypeStruct((1, 1), jnp.float32),
    )(partials.reshape(_NW, _ROWS * _L))
    return out[0, 0]


# speedup vs baseline: 4.4469x; 4.4469x over previous
import jax
import jax.numpy as jnp
import numpy as np
from jax import lax
from jax.experimental import pallas as pl
from jax.experimental.pallas import tpu as pltpu

_BINS = 10
_EDGES32 = np.arange(_BINS + 1, dtype=np.float32) / np.float32(_BINS)
_LOGITS = [float(np.log(np.float64(e) / (1.0 - np.float64(e))))
           for e in _EDGES32[1:_BINS]]
_DOT_DIMS = (((1,), (0,)), ((), ()))


def _ghm_body(x_ref, y_ref, out_ref, acc_ref):
    step = pl.program_id(0)
    nsteps = pl.num_programs(0)

    @pl.when(step == 0)
    def _init():
        acc_ref[...] = jnp.zeros_like(acc_ref)

    x = x_ref[...]
    y = y_ref[...]
    xi = lax.bitcast_convert_type(x, jnp.int32)
    t = lax.bitcast_convert_type(xi ^ (y << 31), jnp.float32)
    e = jnp.exp(-jnp.abs(t))
    pe = jnp.maximum(t, 0.0) + jnp.log1p(e)

    ones = jnp.ones((8, x.shape[0]), jnp.float32)
    rhs = [pe]
    for j in range(1, _BINS):
        m = t >= _LOGITS[j - 1]
        rhs.append(jnp.where(m, pe, 0.0))
    for j in range(1, _BINS):
        m = t >= _LOGITS[j - 1]
        rhs.append(jnp.where(m, 1.0, 0.0))
    for k in range(2 * _BINS - 1):
        d = lax.dot_general(ones, rhs[k], _DOT_DIMS,
                            preferred_element_type=jnp.float32)
        acc_ref[k] = acc_ref[k] + d

    @pl.when(step == nsteps - 1)
    def _finalize():
        total_n = (jnp.float32(x_ref.shape[0] * x_ref.shape[1])
                   * jnp.asarray(nsteps, jnp.float32))
        u = [jnp.sum(acc_ref[k][0, :]) for k in range(_BINS)]
        tt = [total_n] + [jnp.sum(acc_ref[9 + j][0, :])
                          for j in range(1, _BINS)]
        num = jnp.float32(0.0)
        acc = jnp.float32(0.0)
        for b in range(_BINS):
            tb1 = jnp.float32(0.0) if b == _BINS - 1 else tt[b + 1]
            ub1 = jnp.float32(0.0) if b == _BINS - 1 else u[b + 1]
            cnt = tt[b] - tb1
            s = u[b] - ub1
            pos = cnt > 0.0
            num = num + jnp.where(pos, 1.0, 0.0)
            acc = acc + jnp.where(pos, s / jnp.maximum(cnt, 1.0), 0.0)
        out_ref[0, 0] = acc / jnp.maximum(num, 1.0)


def kernel(y_pred, y_true):
    n = y_pred.shape[0]
    cols = 128
    rows = n // cols
    grid = 8
    bm = rows // grid
    x2 = y_pred.reshape(rows, cols)
    y2 = y_true.reshape(rows, cols).astype(jnp.int32)
    out = pl.pallas_call(
        _ghm_body,
        grid=(grid,),
        in_specs=[
            pl.BlockSpec((bm, cols), lambda i: (i, 0)),
            pl.BlockSpec((bm, cols), lambda i: (i, 0)),
        ],
        out_specs=pl.BlockSpec(memory_space=pltpu.SMEM),
        out_shape=jax.ShapeDtypeStruct((1, 1), jnp.float32),
        scratch_shapes=[
            pltpu.VMEM((2 * _BINS - 1, 8, cols), jnp.float32),
        ],
    )(x2, y2)
    return out[0, 0]
